# baseline (speedup 1.0000x reference)
import functools

import jax
import jax.numpy as jnp
from jax import lax
from jax.experimental import pallas as pl
from jax.experimental.pallas import tpu as pltpu
from jax.experimental.pallas import tpu_sc as plsc

DIM = 64
R = 8
SCALING = 2.0

NC = 2
NS = 16
NW = NC * NS
S = 20
XR = 16
NBUF = 3


def _sc_gather(x, weight, lora_a):
    n_rows = x.shape[0]
    rows_pw = n_rows // NW
    n_chunks = rows_pw // XR
    mesh = plsc.VectorSubcoreMesh(core_axis_name="c", subcore_axis_name="s",
                                  num_cores=NC)

    @functools.partial(
        pl.kernel,
        mesh=mesh,
        compiler_params=pltpu.CompilerParams(use_tc_tiling_on_sc=False,
                                             needs_layout_passes=False),
        out_type=[
            jax.ShapeDtypeStruct((n_rows, S, DIM), jnp.float32),
            jax.ShapeDtypeStruct((n_rows, S, R), jnp.float32),
        ],
        scratch_types=[
            pltpu.VMEM((rows_pw, S), jnp.int32),
            pltpu.VMEM((NBUF, XR, S, DIM), jnp.float32),
            pltpu.VMEM((NBUF, XR, S, R), jnp.float32),
            pltpu.SemaphoreType.DMA,
            pltpu.SemaphoreType.DMA,
            pltpu.SemaphoreType.DMA,
            pltpu.SemaphoreType.DMA,
        ],
    )
    def gather_kernel(x_hbm, w_hbm, a_hbm, gw_hbm, ga_hbm,
                      idx_v, wbuf, abuf, sem_w, sem_a, sem_ow, sem_oa):
        cid = lax.axis_index("c")
        sid = lax.axis_index("s")
        wid = sid * NC + cid
        x0 = wid * rows_pw
        pltpu.sync_copy(x_hbm.at[pl.ds(x0, rows_pw)], idx_v)

        def g_copies(c, s):
            cps = []
            for j in range(XR):
                cps.append(pltpu.make_async_copy(
                    w_hbm.at[idx_v.at[c * XR + j]], wbuf.at[s, j], sem_w))
                cps.append(pltpu.make_async_copy(
                    a_hbm.at[idx_v.at[c * XR + j]], abuf.at[s, j], sem_a))
            return cps

        def o_copies(c, s):
            return [
                pltpu.make_async_copy(
                    wbuf.at[s], gw_hbm.at[pl.ds(x0 + c * XR, XR)], sem_ow),
                pltpu.make_async_copy(
                    abuf.at[s], ga_hbm.at[pl.ds(x0 + c * XR, XR)], sem_oa),
            ]

        def step(c, s1, s3):
            for cp in g_copies(c, s1):
                cp.wait()
            for cp in o_copies(c, s1):
                cp.start()

            @pl.when(c + 2 < n_chunks)
            def _():
                @pl.when(c >= 1)
                def _():
                    for cp in o_copies(c - 1, s3):
                        cp.wait()
                for cp in g_copies(c + 2, s3):
                    cp.start()

        for cp in g_copies(0, 0):
            cp.start()
        for cp in g_copies(1, 1):
            cp.start()

        def trio(t, carry):
            for b in range(NBUF):
                step(t * NBUF + b, b, (b + 2) % NBUF)
            return carry

        lax.fori_loop(0, n_chunks // NBUF, trio, 0)
        step(jnp.int32(n_chunks - 2), 0, 2)
        step(jnp.int32(n_chunks - 1), 1, 0)
        for c, s in ((n_chunks - 3, 2), (n_chunks - 2, 0), (n_chunks - 1, 1)):
            for cp in o_copies(c, s):
                cp.wait()

    return gather_kernel(x, weight, lora_a)


def _tc_body(gw_ref, ga_ref, b_ref, out_ref):
    low = jax.lax.dot_general(
        ga_ref[...], b_ref[...] * jnp.float32(SCALING),
        (((2,), (0,)), ((), ())), preferred_element_type=jnp.float32)
    out_ref[...] = gw_ref[...] + low


def _tc_add(gw, ga, lora_b):
    n_rows = gw.shape[0]
    BR = 512
    return pl.pallas_call(
        _tc_body,
        grid=(n_rows // BR,),
        in_specs=[
            pl.BlockSpec((BR, S, DIM), lambda i: (i, 0, 0)),
            pl.BlockSpec((BR, S, R), lambda i: (i, 0, 0)),
            pl.BlockSpec((R, DIM), lambda i: (0, 0)),
        ],
        out_specs=pl.BlockSpec((BR, S, DIM), lambda i: (i, 0, 0)),
        out_shape=jax.ShapeDtypeStruct((n_rows, S, DIM), jnp.float32),
    )(gw, ga, lora_b)


def kernel(x, weight, lora_a, lora_b):
    gw, ga = _sc_gather(x, weight, lora_a)
    return _tc_add(gw, ga, lora_b)
